# baseline (speedup 1.0000x reference)
import functools

import jax
import jax.numpy as jnp
from jax import lax
from jax.experimental import pallas as pl
from jax.experimental.pallas import tpu as pltpu
from jax.experimental.pallas import tpu_sc as plsc

N = 10000
E = 320000
D_IN = 512
D = 128


_ROWS = 2000


def _proj_body(x_ref, wfc_ref, bfc_ref, wg0_ref, wg1_ref, out_ref):
    x = x_ref[...]
    h = jnp.dot(x, wfc_ref[...].T, preferred_element_type=jnp.float32)
    h = h + bfc_ref[...]
    h = jnp.where(h > 0, h, jnp.exp(h) - 1.0)
    out_ref[0] = jnp.dot(h, wg0_ref[...].T, preferred_element_type=jnp.float32)
    out_ref[1] = jnp.dot(h, wg1_ref[...].T, preferred_element_type=jnp.float32)


def _proj(feats0, W_fc, b_fc, W_g0, W_g1):
    return pl.pallas_call(
        _proj_body,
        grid=(N // _ROWS,),
        in_specs=[
            pl.BlockSpec((_ROWS, D_IN), lambda i: (i, 0)),
            pl.BlockSpec((D, D_IN), lambda i: (0, 0)),
            pl.BlockSpec((1, D), lambda i: (0, 0)),
            pl.BlockSpec((D, D), lambda i: (0, 0)),
            pl.BlockSpec((D, D), lambda i: (0, 0)),
        ],
        out_specs=pl.BlockSpec((2, _ROWS, D), lambda i: (0, i, 0)),
        out_shape=jax.ShapeDtypeStruct((2, N, D), jnp.float32),
    )(feats0, W_fc, b_fc.reshape(1, D), W_g0, W_g1)


_CH = 80
_TILES = 16
_EPT = E // _TILES
_NCHUNK = _EPT // _CH
_NPT = 624
_NREM = N - _NPT * _TILES


def _sc_edge_kernel(edges, ew, seq2n):
    mesh = plsc.VectorSubcoreMesh(core_axis_name="c", subcore_axis_name="s")

    @functools.partial(
        pl.kernel,
        mesh=mesh,
        out_type=jax.ShapeDtypeStruct((2, N, D), jnp.float32),
        scratch_types=[
            pltpu.VMEM((3, 2, _CH), jnp.int32),
            pltpu.VMEM((3, _CH + 16), jnp.float32),
            pltpu.VMEM((2, _CH, D), jnp.float32),
            pltpu.VMEM_SHARED((N, D), jnp.float32),
            pltpu.SemaphoreType.DMA((3,)),
            pltpu.SemaphoreType.DMA((2,)),
        ],
    )
    def k(edges_hbm, ew_hbm, seq_hbm, out_hbm, idx_v, ew_v, rows_v, acc,
          semi, semg):
        c = lax.axis_index("c")
        s = lax.axis_index("s")

        def _zrow(k_, carry):
            for j in range(D // 16):
                rows_v[0, k_, pl.ds(16 * j, 16)] = jnp.zeros((16,), jnp.float32)
            return carry
        lax.fori_loop(0, _CH, _zrow, 0)
        nbase = s * _NPT
        for p in range(_NPT // _CH):
            pltpu.sync_copy(rows_v.at[0], acc.at[pl.ds(nbase + p * _CH, _CH)])
        rem = _NPT - (_NPT // _CH) * _CH
        pltpu.sync_copy(rows_v.at[0, pl.ds(0, rem)],
                        acc.at[pl.ds(nbase + (_NPT // _CH) * _CH, rem)])

        @pl.when(s == _TILES - 1)
        def _():
            pltpu.sync_copy(rows_v.at[0, pl.ds(0, _NREM)],
                            acc.at[pl.ds(_NPT * _TILES, _NREM)])

        plsc.subcore_barrier()

        pltpu.sync_copy(edges_hbm.at[c, s, 0], idx_v.at[0])
        pltpu.sync_copy(ew_hbm.at[c, s, 0], ew_v.at[0, pl.ds(0, _CH)])
        pltpu.async_copy(edges_hbm.at[c, s, 1], idx_v.at[1], semi.at[1])
        pltpu.async_copy(ew_hbm.at[c, s, 1], ew_v.at[1, pl.ds(0, _CH)],
                         semi.at[1])
        pltpu.async_copy(seq_hbm.at[idx_v.at[0, 0]], rows_v.at[0], semg.at[0])

        def body(g, carry):
            a = lax.rem(g, 2)
            b = 1 - a
            i0 = lax.rem(g, 3)
            i1 = lax.rem(g + 1, 3)
            i2 = lax.rem(g + 2, 3)
            pltpu.make_async_copy(seq_hbm.at[idx_v.at[i0, 0]],
                                  rows_v.at[a], semg.at[a]).wait()

            @pl.when(g < _NCHUNK - 1)
            def _():
                pltpu.make_async_copy(edges_hbm.at[c, s, g + 1],
                                      idx_v.at[i1], semi.at[i1]).wait()
                pltpu.make_async_copy(ew_hbm.at[c, s, g + 1],
                                      ew_v.at[i1, pl.ds(0, _CH)],
                                      semi.at[i1]).wait()
                pltpu.async_copy(seq_hbm.at[idx_v.at[i1, 0]],
                                 rows_v.at[b], semg.at[b])

            @pl.when(g < _NCHUNK - 2)
            def _():
                pltpu.async_copy(edges_hbm.at[c, s, g + 2],
                                 idx_v.at[i2], semi.at[i2])
                pltpu.async_copy(ew_hbm.at[c, s, g + 2],
                                 ew_v.at[i2, pl.ds(0, _CH)], semi.at[i2])

            @plsc.parallel_loop(0, _CH, unroll=4)
            def _scale(r):
                w = ew_v[i0, pl.ds(r, 16)][0]
                for j in range(D // 16):
                    sl = pl.ds(16 * j, 16)
                    rows_v[a, r, sl] = rows_v[a, r, sl] * w

            pltpu.sync_copy(rows_v.at[a], acc.at[idx_v.at[i0, 1]], add=True)
            return carry

        lax.fori_loop(0, _NCHUNK, body, 0)
        plsc.subcore_barrier()

        pltpu.sync_copy(acc.at[pl.ds(nbase, _NPT)],
                        out_hbm.at[c, pl.ds(nbase, _NPT)])

        @pl.when(s == _TILES - 1)
        def _():
            pltpu.sync_copy(acc.at[pl.ds(_NPT * _TILES, _NREM)],
                            out_hbm.at[c, pl.ds(_NPT * _TILES, _NREM)])

    return k(edges, ew, seq2n)


def _score_body(raw_ref, bg_ref, al_ref, watt_ref, batt_ref, out_ref):
    i = pl.program_id(0)
    parts = []
    for m in range(2):
        x = raw_ref[m] + bg_ref[m]
        e = jnp.where(x > 0, x, al_ref[0, m] * x)
        t = jnp.tanh(jnp.dot(e, watt_ref[...].T,
                             preferred_element_type=jnp.float32) + batt_ref[...])
        parts.append(jnp.sum(t, axis=0, keepdims=True))
    p = jnp.concatenate(parts, axis=0)

    @pl.when(i == 0)
    def _():
        out_ref[...] = p

    @pl.when(i > 0)
    def _():
        out_ref[...] = out_ref[...] + p


def _score(raw, bg, al, W_att, b_att):
    return pl.pallas_call(
        _score_body,
        grid=(N // _ROWS,),
        in_specs=[
            pl.BlockSpec((2, _ROWS, D), lambda i: (0, i, 0)),
            pl.BlockSpec((2, D), lambda i: (0, 0)),
            pl.BlockSpec((1, 2), lambda i: (0, 0)),
            pl.BlockSpec((D, D), lambda i: (0, 0)),
            pl.BlockSpec((1, D), lambda i: (0, 0)),
        ],
        out_specs=pl.BlockSpec((2, D), lambda i: (0, 0)),
        out_shape=jax.ShapeDtypeStruct((2, D), jnp.float32),
    )(raw, bg, al, W_att, b_att)


def _combine_body(raw_ref, bg_ref, al_ref, sums_ref, av_ref, out_ref):
    s0 = jnp.sum(sums_ref[0] * av_ref[0]) / N
    s1 = jnp.sum(sums_ref[1] * av_ref[0]) / N
    m = jnp.maximum(s0, s1)
    e0 = jnp.exp(s0 - m)
    e1 = jnp.exp(s1 - m)
    b0 = e0 / (e0 + e1)
    b1 = e1 / (e0 + e1)
    x0 = raw_ref[0] + bg_ref[0]
    x1 = raw_ref[1] + bg_ref[1]
    p0 = jnp.where(x0 > 0, x0, al_ref[0, 0] * x0)
    p1 = jnp.where(x1 > 0, x1, al_ref[0, 1] * x1)
    out_ref[...] = b0 * p0 + b1 * p1


def _combine(raw, bg, al, sums, att_vec):
    return pl.pallas_call(
        _combine_body,
        grid=(N // _ROWS,),
        in_specs=[
            pl.BlockSpec((2, _ROWS, D), lambda i: (0, i, 0)),
            pl.BlockSpec((2, D), lambda i: (0, 0)),
            pl.BlockSpec((1, 2), lambda i: (0, 0)),
            pl.BlockSpec((2, D), lambda i: (0, 0)),
            pl.BlockSpec((1, D), lambda i: (0, 0)),
        ],
        out_specs=pl.BlockSpec((_ROWS, D), lambda i: (i, 0)),
        out_shape=jax.ShapeDtypeStruct((N, D), jnp.float32),
    )(raw, bg, al, sums, att_vec)


def kernel(feats0, edge_index0, edge_weight0, edge_index1, edge_weight1,
           W_fc, b_fc, W_g0, b_g0, a0, W_g1, b_g1, a1, W_att, b_att, att_vec):
    seq = _proj(feats0, W_fc, b_fc, W_g0, W_g1)
    seq2n = seq.reshape(2 * N, D)
    strip = (2, _TILES, _NCHUNK, _CH)
    src_adj = jnp.concatenate([edge_index0[1], edge_index1[1] + N]).reshape(strip)
    dst_all = jnp.concatenate([edge_index0[0], edge_index1[0]]).reshape(strip)
    edges = jnp.stack([src_adj, dst_all], axis=3)
    ew_all = jnp.concatenate([edge_weight0, edge_weight1]).reshape(strip)
    raw = _sc_edge_kernel(edges, ew_all, seq2n)

    bg = jnp.stack([b_g0, b_g1])
    al = jnp.stack([a0, a1]).reshape(1, 2)
    sums = _score(raw, bg, al, W_att, b_att.reshape(1, D))
    return _combine(raw, bg, al, sums, att_vec)


# speedup vs baseline: 1.1651x; 1.1651x over previous
import functools

import jax
import jax.numpy as jnp
from jax import lax
from jax.experimental import pallas as pl
from jax.experimental.pallas import tpu as pltpu
from jax.experimental.pallas import tpu_sc as plsc

N = 10000
E = 320000
D_IN = 512
D = 128


_ROWS = 2000


def _proj_body(x_ref, wfc_ref, bfc_ref, wg0_ref, wg1_ref, out_ref):
    x = x_ref[...]
    h = jnp.dot(x, wfc_ref[...].T, preferred_element_type=jnp.float32)
    h = h + bfc_ref[...]
    h = jnp.where(h > 0, h, jnp.exp(h) - 1.0)
    out_ref[0] = jnp.dot(h, wg0_ref[...].T, preferred_element_type=jnp.float32)
    out_ref[1] = jnp.dot(h, wg1_ref[...].T, preferred_element_type=jnp.float32)


def _proj(feats0, W_fc, b_fc, W_g0, W_g1):
    return pl.pallas_call(
        _proj_body,
        grid=(N // _ROWS,),
        in_specs=[
            pl.BlockSpec((_ROWS, D_IN), lambda i: (i, 0)),
            pl.BlockSpec((D, D_IN), lambda i: (0, 0)),
            pl.BlockSpec((1, D), lambda i: (0, 0)),
            pl.BlockSpec((D, D), lambda i: (0, 0)),
            pl.BlockSpec((D, D), lambda i: (0, 0)),
        ],
        out_specs=pl.BlockSpec((2, _ROWS, D), lambda i: (0, i, 0)),
        out_shape=jax.ShapeDtypeStruct((2, N, D), jnp.float32),
    )(feats0, W_fc, b_fc.reshape(1, D), W_g0, W_g1)


_CH = 80
_TILES = 16
_EPT = E // _TILES
_NCHUNK = _EPT // _CH
_NPT = 624
_NREM = N - _NPT * _TILES
_RB = 3
_IB = 4


def _sc_edge_kernel(edges, ew, seq2n):
    mesh = plsc.VectorSubcoreMesh(core_axis_name="c", subcore_axis_name="s")

    @functools.partial(
        pl.kernel,
        mesh=mesh,
        out_type=jax.ShapeDtypeStruct((2, N, D), jnp.float32),
        scratch_types=[
            pltpu.VMEM((_IB, 2, _CH), jnp.int32),
            pltpu.VMEM((_IB, _CH + 16), jnp.float32),
            pltpu.VMEM((_RB, _CH, D), jnp.float32),
            pltpu.VMEM_SHARED((N, D), jnp.float32),
            pltpu.SemaphoreType.DMA((_IB,)),
            pltpu.SemaphoreType.DMA((_RB,)),
        ],
    )
    def k(edges_hbm, ew_hbm, seq_hbm, out_hbm, idx_v, ew_v, rows_v, acc,
          semi, semg):
        c = lax.axis_index("c")
        s = lax.axis_index("s")

        def _zrow(k_, carry):
            for j in range(D // 16):
                rows_v[0, k_, pl.ds(16 * j, 16)] = jnp.zeros((16,), jnp.float32)
            return carry
        lax.fori_loop(0, _CH, _zrow, 0)
        nbase = s * _NPT
        for p in range(_NPT // _CH):
            pltpu.sync_copy(rows_v.at[0], acc.at[pl.ds(nbase + p * _CH, _CH)])
        rem = _NPT - (_NPT // _CH) * _CH
        pltpu.sync_copy(rows_v.at[0, pl.ds(0, rem)],
                        acc.at[pl.ds(nbase + (_NPT // _CH) * _CH, rem)])

        @pl.when(s == _TILES - 1)
        def _():
            pltpu.sync_copy(rows_v.at[0, pl.ds(0, _NREM)],
                            acc.at[pl.ds(_NPT * _TILES, _NREM)])

        plsc.subcore_barrier()

        pltpu.sync_copy(edges_hbm.at[c, s, 0], idx_v.at[0])
        pltpu.sync_copy(ew_hbm.at[c, s, 0], ew_v.at[0, pl.ds(0, _CH)])
        pltpu.async_copy(edges_hbm.at[c, s, 1], idx_v.at[1], semi.at[1])
        pltpu.async_copy(ew_hbm.at[c, s, 1], ew_v.at[1, pl.ds(0, _CH)],
                         semi.at[1])
        pltpu.async_copy(edges_hbm.at[c, s, 2], idx_v.at[2], semi.at[2])
        pltpu.async_copy(ew_hbm.at[c, s, 2], ew_v.at[2, pl.ds(0, _CH)],
                         semi.at[2])
        pltpu.async_copy(seq_hbm.at[idx_v.at[0, 0]], rows_v.at[0], semg.at[0])
        pltpu.make_async_copy(edges_hbm.at[c, s, 1], idx_v.at[1],
                              semi.at[1]).wait()
        pltpu.make_async_copy(ew_hbm.at[c, s, 1], ew_v.at[1, pl.ds(0, _CH)],
                              semi.at[1]).wait()
        pltpu.async_copy(seq_hbm.at[idx_v.at[1, 0]], rows_v.at[1], semg.at[1])

        def body(g, carry):
            a = lax.rem(g, _RB)
            a2 = lax.rem(g + 2, _RB)
            i0 = lax.rem(g, _IB)
            i2 = lax.rem(g + 2, _IB)
            i3 = lax.rem(g + 3, _IB)
            pltpu.make_async_copy(seq_hbm.at[idx_v.at[i0, 0]],
                                  rows_v.at[a], semg.at[a]).wait()

            @pl.when(g < _NCHUNK - 2)
            def _():
                pltpu.make_async_copy(edges_hbm.at[c, s, g + 2],
                                      idx_v.at[i2], semi.at[i2]).wait()
                pltpu.make_async_copy(ew_hbm.at[c, s, g + 2],
                                      ew_v.at[i2, pl.ds(0, _CH)],
                                      semi.at[i2]).wait()
                pltpu.async_copy(seq_hbm.at[idx_v.at[i2, 0]],
                                 rows_v.at[a2], semg.at[a2])

            @pl.when(g < _NCHUNK - 3)
            def _():
                pltpu.async_copy(edges_hbm.at[c, s, g + 3],
                                 idx_v.at[i3], semi.at[i3])
                pltpu.async_copy(ew_hbm.at[c, s, g + 3],
                                 ew_v.at[i3, pl.ds(0, _CH)], semi.at[i3])

            @plsc.parallel_loop(0, _CH, unroll=4)
            def _scale(r):
                w = ew_v[i0, pl.ds(r, 16)][0]
                for j in range(D // 16):
                    sl = pl.ds(16 * j, 16)
                    rows_v[a, r, sl] = rows_v[a, r, sl] * w

            pltpu.sync_copy(rows_v.at[a], acc.at[idx_v.at[i0, 1]], add=True)
            return carry

        lax.fori_loop(0, _NCHUNK, body, 0)
        plsc.subcore_barrier()

        pltpu.sync_copy(acc.at[pl.ds(nbase, _NPT)],
                        out_hbm.at[c, pl.ds(nbase, _NPT)])

        @pl.when(s == _TILES - 1)
        def _():
            pltpu.sync_copy(acc.at[pl.ds(_NPT * _TILES, _NREM)],
                            out_hbm.at[c, pl.ds(_NPT * _TILES, _NREM)])

    return k(edges, ew, seq2n)


def _score_body(raw_ref, bg_ref, al_ref, watt_ref, batt_ref, out_ref):
    i = pl.program_id(0)
    parts = []
    for m in range(2):
        x = raw_ref[m] + bg_ref[m]
        e = jnp.where(x > 0, x, al_ref[0, m] * x)
        t = jnp.tanh(jnp.dot(e, watt_ref[...].T,
                             preferred_element_type=jnp.float32) + batt_ref[...])
        parts.append(jnp.sum(t, axis=0, keepdims=True))
    p = jnp.concatenate(parts, axis=0)

    @pl.when(i == 0)
    def _():
        out_ref[...] = p

    @pl.when(i > 0)
    def _():
        out_ref[...] = out_ref[...] + p


def _score(raw, bg, al, W_att, b_att):
    return pl.pallas_call(
        _score_body,
        grid=(N // _ROWS,),
        in_specs=[
            pl.BlockSpec((2, _ROWS, D), lambda i: (0, i, 0)),
            pl.BlockSpec((2, D), lambda i: (0, 0)),
            pl.BlockSpec((1, 2), lambda i: (0, 0)),
            pl.BlockSpec((D, D), lambda i: (0, 0)),
            pl.BlockSpec((1, D), lambda i: (0, 0)),
        ],
        out_specs=pl.BlockSpec((2, D), lambda i: (0, 0)),
        out_shape=jax.ShapeDtypeStruct((2, D), jnp.float32),
    )(raw, bg, al, W_att, b_att)


def _combine_body(raw_ref, bg_ref, al_ref, sums_ref, av_ref, out_ref):
    s0 = jnp.sum(sums_ref[0] * av_ref[0]) / N
    s1 = jnp.sum(sums_ref[1] * av_ref[0]) / N
    m = jnp.maximum(s0, s1)
    e0 = jnp.exp(s0 - m)
    e1 = jnp.exp(s1 - m)
    b0 = e0 / (e0 + e1)
    b1 = e1 / (e0 + e1)
    x0 = raw_ref[0] + bg_ref[0]
    x1 = raw_ref[1] + bg_ref[1]
    p0 = jnp.where(x0 > 0, x0, al_ref[0, 0] * x0)
    p1 = jnp.where(x1 > 0, x1, al_ref[0, 1] * x1)
    out_ref[...] = b0 * p0 + b1 * p1


def _combine(raw, bg, al, sums, att_vec):
    return pl.pallas_call(
        _combine_body,
        grid=(N // _ROWS,),
        in_specs=[
            pl.BlockSpec((2, _ROWS, D), lambda i: (0, i, 0)),
            pl.BlockSpec((2, D), lambda i: (0, 0)),
            pl.BlockSpec((1, 2), lambda i: (0, 0)),
            pl.BlockSpec((2, D), lambda i: (0, 0)),
            pl.BlockSpec((1, D), lambda i: (0, 0)),
        ],
        out_specs=pl.BlockSpec((_ROWS, D), lambda i: (i, 0)),
        out_shape=jax.ShapeDtypeStruct((N, D), jnp.float32),
    )(raw, bg, al, sums, att_vec)


def kernel(feats0, edge_index0, edge_weight0, edge_index1, edge_weight1,
           W_fc, b_fc, W_g0, b_g0, a0, W_g1, b_g1, a1, W_att, b_att, att_vec):
    seq = _proj(feats0, W_fc, b_fc, W_g0, W_g1)
    seq2n = seq.reshape(2 * N, D)
    strip = (2, _TILES, _NCHUNK, _CH)
    src_adj = jnp.concatenate([edge_index0[1], edge_index1[1] + N]).reshape(strip)
    dst_all = jnp.concatenate([edge_index0[0], edge_index1[0]]).reshape(strip)
    edges = jnp.stack([src_adj, dst_all], axis=3)
    ew_all = jnp.concatenate([edge_weight0, edge_weight1]).reshape(strip)
    raw = _sc_edge_kernel(edges, ew_all, seq2n)

    bg = jnp.stack([b_g0, b_g1])
    al = jnp.stack([a0, a1]).reshape(1, 2)
    sums = _score(raw, bg, al, W_att, b_att.reshape(1, D))
    return _combine(raw, bg, al, sums, att_vec)


# speedup vs baseline: 1.1743x; 1.0078x over previous
import functools

import jax
import jax.numpy as jnp
from jax import lax
from jax.experimental import pallas as pl
from jax.experimental.pallas import tpu as pltpu
from jax.experimental.pallas import tpu_sc as plsc

N = 10000
E = 320000
D_IN = 512
D = 128


_ROWS = 2000


def _proj_body(x_ref, wfc_ref, bfc_ref, wg0_ref, wg1_ref, out_ref):
    x = x_ref[...]
    h = jnp.dot(x, wfc_ref[...].T, preferred_element_type=jnp.float32)
    h = h + bfc_ref[...]
    h = jnp.where(h > 0, h, jnp.exp(h) - 1.0)
    out_ref[0] = jnp.dot(h, wg0_ref[...].T, preferred_element_type=jnp.float32)
    out_ref[1] = jnp.dot(h, wg1_ref[...].T, preferred_element_type=jnp.float32)


def _proj(feats0, W_fc, b_fc, W_g0, W_g1):
    return pl.pallas_call(
        _proj_body,
        grid=(N // _ROWS,),
        in_specs=[
            pl.BlockSpec((_ROWS, D_IN), lambda i: (i, 0)),
            pl.BlockSpec((D, D_IN), lambda i: (0, 0)),
            pl.BlockSpec((1, D), lambda i: (0, 0)),
            pl.BlockSpec((D, D), lambda i: (0, 0)),
            pl.BlockSpec((D, D), lambda i: (0, 0)),
        ],
        out_specs=pl.BlockSpec((2, _ROWS, D), lambda i: (0, i, 0)),
        out_shape=jax.ShapeDtypeStruct((2, N, D), jnp.float32),
    )(feats0, W_fc, b_fc.reshape(1, D), W_g0, W_g1)


_CH = 80
_TILES = 16
_EPT = E // _TILES
_NCHUNK = _EPT // _CH
_NPT = 624
_NREM = N - _NPT * _TILES
_RB = 4
_IB = 6


def _sc_edge_kernel(edges, ew, seq2n):
    mesh = plsc.VectorSubcoreMesh(core_axis_name="c", subcore_axis_name="s")

    @functools.partial(
        pl.kernel,
        mesh=mesh,
        out_type=jax.ShapeDtypeStruct((2, N, D), jnp.float32),
        scratch_types=[
            pltpu.VMEM((_IB, 2, _CH), jnp.int32),
            pltpu.VMEM((_IB, _CH + 16), jnp.float32),
            pltpu.VMEM((_RB, _CH, D), jnp.float32),
            pltpu.VMEM_SHARED((N, D), jnp.float32),
            pltpu.SemaphoreType.DMA((_IB,)),
            pltpu.SemaphoreType.DMA((_RB,)),
        ],
    )
    def k(edges_hbm, ew_hbm, seq_hbm, out_hbm, idx_v, ew_v, rows_v, acc,
          semi, semg):
        c = lax.axis_index("c")
        s = lax.axis_index("s")

        def _zrow(k_, carry):
            for j in range(D // 16):
                rows_v[0, k_, pl.ds(16 * j, 16)] = jnp.zeros((16,), jnp.float32)
            return carry
        lax.fori_loop(0, _CH, _zrow, 0)
        nbase = s * _NPT
        for p in range(_NPT // _CH):
            pltpu.sync_copy(rows_v.at[0], acc.at[pl.ds(nbase + p * _CH, _CH)])
        rem = _NPT - (_NPT // _CH) * _CH
        pltpu.sync_copy(rows_v.at[0, pl.ds(0, rem)],
                        acc.at[pl.ds(nbase + (_NPT // _CH) * _CH, rem)])

        @pl.when(s == _TILES - 1)
        def _():
            pltpu.sync_copy(rows_v.at[0, pl.ds(0, _NREM)],
                            acc.at[pl.ds(_NPT * _TILES, _NREM)])

        plsc.subcore_barrier()

        pltpu.sync_copy(edges_hbm.at[c, s, 0], idx_v.at[0])
        pltpu.sync_copy(ew_hbm.at[c, s, 0], ew_v.at[0, pl.ds(0, _CH)])
        pltpu.async_copy(edges_hbm.at[c, s, 1], idx_v.at[1], semi.at[1])
        pltpu.async_copy(ew_hbm.at[c, s, 1], ew_v.at[1, pl.ds(0, _CH)],
                         semi.at[1])
        pltpu.async_copy(edges_hbm.at[c, s, 2], idx_v.at[2], semi.at[2])
        pltpu.async_copy(ew_hbm.at[c, s, 2], ew_v.at[2, pl.ds(0, _CH)],
                         semi.at[2])
        pltpu.async_copy(seq_hbm.at[idx_v.at[0, 0]], rows_v.at[0], semg.at[0])
        pltpu.make_async_copy(edges_hbm.at[c, s, 1], idx_v.at[1],
                              semi.at[1]).wait()
        pltpu.make_async_copy(ew_hbm.at[c, s, 1], ew_v.at[1, pl.ds(0, _CH)],
                              semi.at[1]).wait()
        pltpu.async_copy(seq_hbm.at[idx_v.at[1, 0]], rows_v.at[1], semg.at[1])
        pltpu.async_copy(edges_hbm.at[c, s, 3], idx_v.at[3], semi.at[3])
        pltpu.async_copy(ew_hbm.at[c, s, 3], ew_v.at[3, pl.ds(0, _CH)],
                         semi.at[3])
        pltpu.make_async_copy(edges_hbm.at[c, s, 2], idx_v.at[2],
                              semi.at[2]).wait()
        pltpu.make_async_copy(ew_hbm.at[c, s, 2], ew_v.at[2, pl.ds(0, _CH)],
                              semi.at[2]).wait()
        pltpu.async_copy(seq_hbm.at[idx_v.at[2, 0]], rows_v.at[2], semg.at[2])

        def body(g, carry):
            a = lax.rem(g, _RB)
            a3 = lax.rem(g + 3, _RB)
            i0 = lax.rem(g, _IB)
            i3 = lax.rem(g + 3, _IB)
            i4 = lax.rem(g + 4, _IB)
            pltpu.make_async_copy(seq_hbm.at[idx_v.at[i0, 0]],
                                  rows_v.at[a], semg.at[a]).wait()

            @pl.when(g < _NCHUNK - 3)
            def _():
                pltpu.make_async_copy(edges_hbm.at[c, s, g + 3],
                                      idx_v.at[i3], semi.at[i3]).wait()
                pltpu.make_async_copy(ew_hbm.at[c, s, g + 3],
                                      ew_v.at[i3, pl.ds(0, _CH)],
                                      semi.at[i3]).wait()
                pltpu.async_copy(seq_hbm.at[idx_v.at[i3, 0]],
                                 rows_v.at[a3], semg.at[a3])

            @pl.when(g < _NCHUNK - 4)
            def _():
                pltpu.async_copy(edges_hbm.at[c, s, g + 4],
                                 idx_v.at[i4], semi.at[i4])
                pltpu.async_copy(ew_hbm.at[c, s, g + 4],
                                 ew_v.at[i4, pl.ds(0, _CH)], semi.at[i4])

            @plsc.parallel_loop(0, _CH, unroll=4)
            def _scale(r):
                w = ew_v[i0, pl.ds(r, 16)][0]
                for j in range(D // 16):
                    sl = pl.ds(16 * j, 16)
                    rows_v[a, r, sl] = rows_v[a, r, sl] * w

            pltpu.sync_copy(rows_v.at[a], acc.at[idx_v.at[i0, 1]], add=True)
            return carry

        lax.fori_loop(0, _NCHUNK, body, 0)
        plsc.subcore_barrier()

        pltpu.sync_copy(acc.at[pl.ds(nbase, _NPT)],
                        out_hbm.at[c, pl.ds(nbase, _NPT)])

        @pl.when(s == _TILES - 1)
        def _():
            pltpu.sync_copy(acc.at[pl.ds(_NPT * _TILES, _NREM)],
                            out_hbm.at[c, pl.ds(_NPT * _TILES, _NREM)])

    return k(edges, ew, seq2n)


def _score_body(raw_ref, bg_ref, al_ref, watt_ref, batt_ref, out_ref):
    i = pl.program_id(0)
    parts = []
    for m in range(2):
        x = raw_ref[m] + bg_ref[m]
        e = jnp.where(x > 0, x, al_ref[0, m] * x)
        t = jnp.tanh(jnp.dot(e, watt_ref[...].T,
                             preferred_element_type=jnp.float32) + batt_ref[...])
        parts.append(jnp.sum(t, axis=0, keepdims=True))
    p = jnp.concatenate(parts, axis=0)

    @pl.when(i == 0)
    def _():
        out_ref[...] = p

    @pl.when(i > 0)
    def _():
        out_ref[...] = out_ref[...] + p


def _score(raw, bg, al, W_att, b_att):
    return pl.pallas_call(
        _score_body,
        grid=(N // _ROWS,),
        in_specs=[
            pl.BlockSpec((2, _ROWS, D), lambda i: (0, i, 0)),
            pl.BlockSpec((2, D), lambda i: (0, 0)),
            pl.BlockSpec((1, 2), lambda i: (0, 0)),
            pl.BlockSpec((D, D), lambda i: (0, 0)),
            pl.BlockSpec((1, D), lambda i: (0, 0)),
        ],
        out_specs=pl.BlockSpec((2, D), lambda i: (0, 0)),
        out_shape=jax.ShapeDtypeStruct((2, D), jnp.float32),
    )(raw, bg, al, W_att, b_att)


def _combine_body(raw_ref, bg_ref, al_ref, sums_ref, av_ref, out_ref):
    s0 = jnp.sum(sums_ref[0] * av_ref[0]) / N
    s1 = jnp.sum(sums_ref[1] * av_ref[0]) / N
    m = jnp.maximum(s0, s1)
    e0 = jnp.exp(s0 - m)
    e1 = jnp.exp(s1 - m)
    b0 = e0 / (e0 + e1)
    b1 = e1 / (e0 + e1)
    x0 = raw_ref[0] + bg_ref[0]
    x1 = raw_ref[1] + bg_ref[1]
    p0 = jnp.where(x0 > 0, x0, al_ref[0, 0] * x0)
    p1 = jnp.where(x1 > 0, x1, al_ref[0, 1] * x1)
    out_ref[...] = b0 * p0 + b1 * p1


def _combine(raw, bg, al, sums, att_vec):
    return pl.pallas_call(
        _combine_body,
        grid=(N // _ROWS,),
        in_specs=[
            pl.BlockSpec((2, _ROWS, D), lambda i: (0, i, 0)),
            pl.BlockSpec((2, D), lambda i: (0, 0)),
            pl.BlockSpec((1, 2), lambda i: (0, 0)),
            pl.BlockSpec((2, D), lambda i: (0, 0)),
            pl.BlockSpec((1, D), lambda i: (0, 0)),
        ],
        out_specs=pl.BlockSpec((_ROWS, D), lambda i: (i, 0)),
        out_shape=jax.ShapeDtypeStruct((N, D), jnp.float32),
    )(raw, bg, al, sums, att_vec)


def kernel(feats0, edge_index0, edge_weight0, edge_index1, edge_weight1,
           W_fc, b_fc, W_g0, b_g0, a0, W_g1, b_g1, a1, W_att, b_att, att_vec):
    seq = _proj(feats0, W_fc, b_fc, W_g0, W_g1)
    seq2n = seq.reshape(2 * N, D)
    strip = (2, _TILES, _NCHUNK, _CH)
    src_adj = jnp.concatenate([edge_index0[1], edge_index1[1] + N]).reshape(strip)
    dst_all = jnp.concatenate([edge_index0[0], edge_index1[0]]).reshape(strip)
    edges = jnp.stack([src_adj, dst_all], axis=3)
    ew_all = jnp.concatenate([edge_weight0, edge_weight1]).reshape(strip)
    raw = _sc_edge_kernel(edges, ew_all, seq2n)

    bg = jnp.stack([b_g0, b_g1])
    al = jnp.stack([a0, a1]).reshape(1, 2)
    sums = _score(raw, bg, al, W_att, b_att.reshape(1, D))
    return _combine(raw, bg, al, sums, att_vec)
